# baseline (speedup 1.0000x reference)
import functools

import jax
import jax.numpy as jnp
from jax import lax
from jax.experimental import pallas as pl
from jax.experimental.pallas import tpu as pltpu
from jax.experimental.pallas import tpu_sc as plsc

BATCH = 4096
SEQ = 200
EMBED_DIM = 64
NUM_CORES = 2
NUM_SUBCORES = 16
NW = NUM_CORES * NUM_SUBCORES
BT = BATCH // NW
CT = EMBED_DIM // 8
SB = SEQ // 8

_mesh = plsc.VectorSubcoreMesh(core_axis_name="c", subcore_axis_name="s")


@functools.partial(
    pl.kernel,
    out_type=jax.ShapeDtypeStruct((SEQ, CT, NW, 8, BT), jnp.float32),
    mesh=_mesh,
    compiler_params=pltpu.CompilerParams(
        use_tc_tiling_on_sc=True, needs_layout_passes=False),
    scratch_types=[
        pltpu.VMEM((SB, 8, BT), jnp.int32),
        pltpu.VMEM((SB, 8, BT), jnp.int32),
        pltpu.VMEM((BT, 128), jnp.float32),
        pltpu.VMEM((BT, 128), jnp.float32),
        pltpu.VMEM((BT, 128), jnp.float32),
        pltpu.VMEM((CT, 8, BT), jnp.float32),
        pltpu.VMEM((CT, 8, BT), jnp.float32),
        pltpu.SemaphoreType.DMA,
        pltpu.SemaphoreType.DMA,
        pltpu.SemaphoreType.DMA,
        pltpu.SemaphoreType.DMA,
        pltpu.SemaphoreType.DMA,
    ],
)
def _sc_fused(idx_hbm, table_hbm, out_hbm, idx_v, par_v, rows0, rows1, rows2,
              t0, t1, gs0, gs1, gs2, ws0, ws1):
    wid = lax.axis_index("s") * NUM_CORES + lax.axis_index("c")
    rows = (rows0, rows1, rows2)
    tbuf = (t0, t1)
    gsem = (gs0, gs1, gs2)
    wsem = (ws0, ws1)

    pltpu.sync_copy(idx_hbm.at[:, wid], idx_v)
    iota = lax.iota(jnp.int32, 16)

    @plsc.parallel_loop(0, SB * 8 * BT // 16, 1, unroll=8)
    def _split(j):
        st = j >> 6
        si = (j >> 3) & 7
        off = (j & 7) * 16
        v = idx_v[st, si, pl.ds(off, 16)]
        idx_v[st, si, pl.ds(off, 16)] = v >> 1
        par_v[st, si, pl.ds(off, 16)] = (v & 1) << 6

    def start_gather(s, b):
        pltpu.async_copy(table_hbm.at[idx_v.at[s >> 3, s & 7]], rows[b], gsem[b])

    def wait_gather(s, b):
        pltpu.make_async_copy(
            table_hbm.at[idx_v.at[s >> 3, s & 7]], rows[b], gsem[b]).wait()

    def start_write(s, b):
        pltpu.async_copy(tbuf[b], out_hbm.at[s, :, wid], wsem[b])

    def wait_write(s, b):
        pltpu.make_async_copy(tbuf[b], out_hbm.at[s, :, wid], wsem[b]).wait()

    rowvs = [iota + bg * 16 for bg in range(BT // 16)]

    def transpose(s, gb, tb):
        colbase = [par_v[s >> 3, s & 7, pl.ds(bg * 16, 16)]
                   for bg in range(BT // 16)]

        @plsc.parallel_loop(0, CT * 8, 1, unroll=2)
        def _t(j):
            ci = j & 7
            ct = j >> 3
            for bg in range(BT // 16):
                v = plsc.load_gather(rows[gb], [rowvs[bg], colbase[bg] + j])
                tbuf[tb][ct, ci, pl.ds(bg * 16, 16)] = v

    for b in range(3):
        start_gather(b, b)

    def body(i, _):
        s = 6 * i
        for k in range(6):
            sb = s + k
            gb = k % 3
            tb = k % 2

            @pl.when(sb >= 2)
            def _():
                wait_write(sb - 2, tb)

            wait_gather(sb, gb)
            transpose(sb, gb, tb)

            @pl.when(sb + 3 < SEQ)
            def _():
                start_gather(sb + 3, gb)

            start_write(sb, tb)
        return _

    lax.fori_loop(0, SEQ // 6, body, None)
    for sb in (198, 199):
        gb = sb % 3
        tb = sb % 2
        wait_write(sb - 2, tb)
        wait_gather(sb, gb)
        transpose(sb, gb, tb)
        start_write(sb, tb)
    wait_write(198, 0)
    wait_write(199, 1)


def kernel(input_ids, word_embeddings):
    idx = (input_ids.astype(jnp.int32).T
           .reshape(SB, 8, NW, BT).transpose(0, 2, 1, 3))
    table2 = word_embeddings.reshape(500000, 128)
    out5 = _sc_fused(idx, table2)
    return out5.transpose(2, 4, 0, 1, 3).reshape(BATCH, SEQ, EMBED_DIM)


# speedup vs baseline: 1.0014x; 1.0014x over previous
import functools

import jax
import jax.numpy as jnp
from jax import lax
from jax.experimental import pallas as pl
from jax.experimental.pallas import tpu as pltpu
from jax.experimental.pallas import tpu_sc as plsc

BATCH = 4096
SEQ = 200
EMBED_DIM = 64
NUM_CORES = 2
NUM_SUBCORES = 16
NW = NUM_CORES * NUM_SUBCORES
BT = BATCH // NW
CT = EMBED_DIM // 8
SB = SEQ // 8

_mesh = plsc.VectorSubcoreMesh(core_axis_name="c", subcore_axis_name="s")


@functools.partial(
    pl.kernel,
    out_type=jax.ShapeDtypeStruct((SEQ, CT, NW, 8, BT), jnp.float32),
    mesh=_mesh,
    compiler_params=pltpu.CompilerParams(
        use_tc_tiling_on_sc=True, needs_layout_passes=False),
    scratch_types=[
        pltpu.VMEM((SB, 8, BT), jnp.int32),
        pltpu.VMEM((SB, 8, BT), jnp.int32),
        pltpu.VMEM((BT, 128), jnp.float32),
        pltpu.VMEM((BT, 128), jnp.float32),
        pltpu.VMEM((BT, 128), jnp.float32),
        pltpu.VMEM((CT, 8, BT), jnp.float32),
        pltpu.VMEM((CT, 8, BT), jnp.float32),
        pltpu.SemaphoreType.DMA,
        pltpu.SemaphoreType.DMA,
        pltpu.SemaphoreType.DMA,
        pltpu.SemaphoreType.DMA,
        pltpu.SemaphoreType.DMA,
    ],
)
def _sc_fused(idx_hbm, table_hbm, out_hbm, idx_v, par_v, rows0, rows1, rows2,
              t0, t1, gs0, gs1, gs2, ws0, ws1):
    wid = lax.axis_index("s") * NUM_CORES + lax.axis_index("c")
    rows = (rows0, rows1, rows2)
    tbuf = (t0, t1)
    gsem = (gs0, gs1, gs2)
    wsem = (ws0, ws1)

    pltpu.sync_copy(idx_hbm.at[:, wid], idx_v)
    iota = lax.iota(jnp.int32, 16)

    @plsc.parallel_loop(0, SB * 8 * BT // 16, 1, unroll=8)
    def _split(j):
        st = j >> 6
        si = (j >> 3) & 7
        off = (j & 7) * 16
        v = idx_v[st, si, pl.ds(off, 16)]
        idx_v[st, si, pl.ds(off, 16)] = v >> 1
        par_v[st, si, pl.ds(off, 16)] = (v & 1) << 6

    def start_gather(s, b):
        pltpu.async_copy(table_hbm.at[idx_v.at[s >> 3, s & 7]], rows[b], gsem[b])

    def wait_gather(s, b):
        pltpu.make_async_copy(
            table_hbm.at[idx_v.at[s >> 3, s & 7]], rows[b], gsem[b]).wait()

    def start_write(s, b):
        pltpu.async_copy(tbuf[b], out_hbm.at[s, :, wid], wsem[b])

    def wait_write(s, b):
        pltpu.make_async_copy(tbuf[b], out_hbm.at[s, :, wid], wsem[b]).wait()

    rowvs = [iota + bg * 16 for bg in range(BT // 16)]

    def transpose(s, gb, tb):
        colbase = [par_v[s >> 3, s & 7, pl.ds(bg * 16, 16)]
                   for bg in range(BT // 16)]

        @plsc.parallel_loop(0, CT * 8, 1, unroll=4)
        def _t(j):
            ci = j & 7
            ct = j >> 3
            for bg in range(BT // 16):
                v = plsc.load_gather(rows[gb], [rowvs[bg], colbase[bg] + j])
                tbuf[tb][ct, ci, pl.ds(bg * 16, 16)] = v

    for b in range(3):
        start_gather(b, b)

    def body(i, _):
        s = 6 * i
        for k in range(6):
            sb = s + k
            gb = k % 3
            tb = k % 2

            @pl.when(sb >= 2)
            def _():
                wait_write(sb - 2, tb)

            wait_gather(sb, gb)
            transpose(sb, gb, tb)

            @pl.when(sb + 3 < SEQ)
            def _():
                start_gather(sb + 3, gb)

            start_write(sb, tb)
        return _

    lax.fori_loop(0, SEQ // 6, body, None)
    for sb in (198, 199):
        gb = sb % 3
        tb = sb % 2
        wait_write(sb - 2, tb)
        wait_gather(sb, gb)
        transpose(sb, gb, tb)
        start_write(sb, tb)
    wait_write(198, 0)
    wait_write(199, 1)


def kernel(input_ids, word_embeddings):
    idx = (input_ids.astype(jnp.int32).T
           .reshape(SB, 8, NW, BT).transpose(0, 2, 1, 3))
    table2 = word_embeddings.reshape(500000, 128)
    out5 = _sc_fused(idx, table2)
    return out5.transpose(2, 4, 0, 1, 3).reshape(BATCH, SEQ, EMBED_DIM)


# speedup vs baseline: 1.0200x; 1.0186x over previous
import functools

import jax
import jax.numpy as jnp
from jax import lax
from jax.experimental import pallas as pl
from jax.experimental.pallas import tpu as pltpu
from jax.experimental.pallas import tpu_sc as plsc

BATCH = 4096
SEQ = 200
EMBED_DIM = 64
TOTAL = BATCH * SEQ
NUM_CORES = 2
NUM_SUBCORES = 16
NW = NUM_CORES * NUM_SUBCORES
B_PER_W = TOTAL // NW
IDXV = 128
KSUB = 4
CHUNK = IDXV * KSUB
NSTEPS = B_PER_W // CHUNK

_mesh = plsc.VectorSubcoreMesh(core_axis_name="c", subcore_axis_name="s")


@functools.partial(
    pl.kernel,
    out_type=jax.ShapeDtypeStruct((TOTAL, EMBED_DIM), jnp.float32),
    mesh=_mesh,
    compiler_params=pltpu.CompilerParams(use_tc_tiling_on_sc=False),
    scratch_types=[
        pltpu.VMEM((NSTEPS * KSUB, IDXV), jnp.int32),
        pltpu.VMEM((CHUNK, EMBED_DIM), jnp.float32),
        pltpu.VMEM((CHUNK, EMBED_DIM), jnp.float32),
        pltpu.SemaphoreType.DMA,
        pltpu.SemaphoreType.DMA,
    ],
)
def _sc_gather(idx_hbm, table_hbm, out_hbm, idx_v, rows0, rows1, sem0, sem1):
    wid = lax.axis_index("s") * NUM_CORES + lax.axis_index("c")
    base = wid * B_PER_W

    pltpu.sync_copy(idx_hbm.at[wid], idx_v)

    rows = (rows0, rows1)
    sems = (sem0, sem1)

    def _start(step, b):
        for j in range(KSUB):
            pltpu.async_copy(
                table_hbm.at[idx_v.at[step * KSUB + j]],
                rows[b].at[pl.ds(j * IDXV, IDXV)],
                sems[b],
            )

    def _finish(step, b):
        for j in range(KSUB):
            pltpu.make_async_copy(
                table_hbm.at[idx_v.at[step * KSUB + j]],
                rows[b].at[pl.ds(j * IDXV, IDXV)],
                sems[b],
            ).wait()
        pltpu.sync_copy(rows[b], out_hbm.at[pl.ds(base + step * CHUNK, CHUNK)])

    _start(0, 0)

    def body(i, _):
        g = 2 * i
        _start(g + 1, 1)
        _finish(g, 0)

        @pl.when(g + 2 < NSTEPS)
        def _():
            _start(g + 2, 0)

        _finish(g + 1, 1)
        return _

    lax.fori_loop(0, NSTEPS // 2, body, None)


def kernel(input_ids, word_embeddings):
    idx = input_ids.astype(jnp.int32).reshape(NW, NSTEPS * KSUB, IDXV)
    out = _sc_gather(idx, word_embeddings)
    return out.reshape(BATCH, SEQ, EMBED_DIM)
